# baseline (speedup 1.0000x reference)
import functools

import jax
import jax.numpy as jnp
from jax import lax
from jax.experimental import pallas as pl
from jax.experimental.pallas import tpu as pltpu
from jax.experimental.pallas import tpu_sc as plsc

_NC = 2
_NS = 16
_NW = _NC * _NS
_CHUNK = 128
_NBUF = 4


@functools.lru_cache(maxsize=None)
def _make_gather(B, D):
    b_per_w = B // _NW
    nchunks = b_per_w // _CHUNK
    mesh = plsc.VectorSubcoreMesh(
        core_axis_name="c", subcore_axis_name="s",
        num_cores=_NC, num_subcores=_NS)

    scratch = (
        [pltpu.VMEM((nchunks, _CHUNK), jnp.int32)]
        + [pltpu.VMEM((_CHUNK, D), jnp.float32) for _ in range(_NBUF)]
        + [pltpu.SemaphoreType.DMA for _ in range(2 * _NBUF)]
    )

    @functools.partial(
        pl.kernel,
        out_type=jax.ShapeDtypeStruct((B, D), jnp.float32),
        mesh=mesh,
        scratch_types=scratch,
        compiler_params=pltpu.CompilerParams(use_tc_tiling_on_sc=False),
    )
    def gather_kernel(idx_hbm, table_hbm, out_hbm, idx_v, *rest):
        rows = rest[:_NBUF]
        gsem = rest[_NBUF:2 * _NBUF]
        wsem = rest[2 * _NBUF:]
        wid = lax.axis_index("s") * _NC + lax.axis_index("c")
        base = wid * b_per_w

        pltpu.sync_copy(idx_hbm.at[wid], idx_v)

        for b in range(_NBUF):
            pltpu.async_copy(table_hbm.at[idx_v.at[b]], rows[b], gsem[b])

        @pl.loop(0, nchunks, step=_NBUF)
        def _(g):
            for b in range(_NBUF):
                j = g + b
                pltpu.make_async_copy(
                    table_hbm.at[idx_v.at[j]], rows[b], gsem[b]).wait()
                dst = out_hbm.at[pl.ds(base + j * _CHUNK, _CHUNK)]
                pltpu.async_copy(rows[b], dst, wsem[b])

                @pl.when(j + _NBUF < nchunks)
                def _():
                    pltpu.make_async_copy(rows[b], dst, wsem[b]).wait()
                    pltpu.async_copy(
                        table_hbm.at[idx_v.at[j + _NBUF]], rows[b], gsem[b])

        for b in range(_NBUF):
            j = nchunks - _NBUF + b
            pltpu.make_async_copy(
                rows[b],
                out_hbm.at[pl.ds(base + j * _CHUNK, _CHUNK)],
                wsem[b]).wait()

    return gather_kernel


def kernel(text, embedding_weight):
    Bt, L = text.shape
    V, D = embedding_weight.shape
    B = Bt * L
    idx = text.reshape(_NW, (B // _NW) // _CHUNK, _CHUNK).astype(jnp.int32)
    out = _make_gather(B, D)(idx, embedding_weight)
    return out.reshape(Bt, 1, L, D)
